# baseline (device time: 13996 ns/iter reference)
import jax
import jax.numpy as jnp
from jax import lax
from jax.experimental import pallas as pl
from jax.experimental.pallas import tpu as pltpu

C = 1


def kernel(partial, resid, gamma):
    _, m, d = partial.shape
    rows = m // C
    gamma2 = gamma.reshape(1, d)

    def body(p_ref, r_ref, g_ref, o_ref, send_ref, comm_ref,
             sscale_ref, cscale_ref, send_sems, recv_sems,
             ssc_sem, rsc_sem):
        my_x = lax.axis_index("x")
        my_y = lax.axis_index("y")
        my_z = lax.axis_index("z")
        nbr = (1 - my_x, my_y, my_z)

        p = p_ref[0]
        amax = jnp.max(jnp.abs(p), axis=-1, keepdims=True)
        scale = jnp.maximum(amax, 1e-20) * (1.0 / 127.0)
        sscale_ref[...] = scale
        send_ref[...] = jnp.round(p / scale).astype(jnp.int8)

        barrier_sem = pltpu.get_barrier_semaphore()
        pl.semaphore_signal(
            barrier_sem, inc=1, device_id=nbr,
            device_id_type=pl.DeviceIdType.MESH,
        )
        pl.semaphore_wait(barrier_sem, 1)

        sc_rdma = pltpu.make_async_remote_copy(
            src_ref=sscale_ref,
            dst_ref=cscale_ref,
            send_sem=ssc_sem,
            recv_sem=rsc_sem,
            device_id=nbr,
            device_id_type=pl.DeviceIdType.MESH,
        )
        sc_rdma.start()
        rdmas = []
        for c in range(C):
            sl = slice(c * rows, (c + 1) * rows)
            rdma = pltpu.make_async_remote_copy(
                src_ref=send_ref.at[sl],
                dst_ref=comm_ref.at[sl],
                send_sem=send_sems.at[c],
                recv_sem=recv_sems.at[c],
                device_id=nbr,
                device_id_type=pl.DeviceIdType.MESH,
            )
            rdma.start()
            rdmas.append(rdma)

        sc_rdma.wait_recv()
        for c in range(C):
            rdmas[c].wait_recv()
            sl = slice(c * rows, (c + 1) * rows)
            theirs = comm_ref[sl, :].astype(jnp.float32) * cscale_ref[sl, :]
            y = p_ref[0, sl, :] + theirs + r_ref[sl, :]
            rms = jnp.sqrt(jnp.mean(y * y, axis=-1, keepdims=True) + 1e-6)
            o_ref[sl, :] = y / rms * g_ref[...]

        sc_rdma.wait_send()
        for c in range(C):
            rdmas[c].wait_send()

    return pl.pallas_call(
        body,
        out_shape=jax.ShapeDtypeStruct((m, d), jnp.float32),
        in_specs=[
            pl.BlockSpec(memory_space=pltpu.VMEM),
            pl.BlockSpec(memory_space=pltpu.VMEM),
            pl.BlockSpec(memory_space=pltpu.VMEM),
        ],
        out_specs=pl.BlockSpec(memory_space=pltpu.VMEM),
        scratch_shapes=[
            pltpu.VMEM((m, d), jnp.int8),
            pltpu.VMEM((m, d), jnp.int8),
            pltpu.VMEM((m, 1), jnp.float32),
            pltpu.VMEM((m, 1), jnp.float32),
            pltpu.SemaphoreType.DMA((C,)),
            pltpu.SemaphoreType.DMA((C,)),
            pltpu.SemaphoreType.DMA,
            pltpu.SemaphoreType.DMA,
        ],
        compiler_params=pltpu.CompilerParams(collective_id=0),
    )(partial, resid, gamma2)


# device time: 13588 ns/iter; 1.0300x vs baseline; 1.0300x over previous
import jax
import jax.numpy as jnp
from jax import lax
from jax.experimental import pallas as pl
from jax.experimental.pallas import tpu as pltpu

C = 4


def kernel(partial, resid, gamma):
    _, m, d = partial.shape
    rows = m // C
    gamma2 = gamma.reshape(1, d)

    def body(p_ref, r_ref, g_ref, o_ref, send_ref, comm_ref,
             send_sems, recv_sems):
        my_x = lax.axis_index("x")
        my_y = lax.axis_index("y")
        my_z = lax.axis_index("z")
        nbr = (1 - my_x, my_y, my_z)

        send_ref[...] = p_ref[0].astype(jnp.bfloat16)

        barrier_sem = pltpu.get_barrier_semaphore()
        pl.semaphore_signal(
            barrier_sem, inc=1, device_id=nbr,
            device_id_type=pl.DeviceIdType.MESH,
        )
        pl.semaphore_wait(barrier_sem, 1)

        rdmas = []
        for c in range(C):
            sl = slice(c * rows, (c + 1) * rows)
            rdma = pltpu.make_async_remote_copy(
                src_ref=send_ref.at[sl],
                dst_ref=comm_ref.at[sl],
                send_sem=send_sems.at[c],
                recv_sem=recv_sems.at[c],
                device_id=nbr,
                device_id_type=pl.DeviceIdType.MESH,
            )
            rdma.start()
            rdmas.append(rdma)

        for c in range(C):
            rdmas[c].wait_recv()
            sl = slice(c * rows, (c + 1) * rows)
            y = (p_ref[0, sl, :]
                 + comm_ref[sl, :].astype(jnp.float32)
                 + r_ref[sl, :])
            rms = jnp.sqrt(jnp.mean(y * y, axis=-1, keepdims=True) + 1e-6)
            o_ref[sl, :] = y / rms * g_ref[...]

        for c in range(C):
            rdmas[c].wait_send()

    return pl.pallas_call(
        body,
        out_shape=jax.ShapeDtypeStruct((m, d), jnp.float32),
        in_specs=[
            pl.BlockSpec(memory_space=pltpu.VMEM),
            pl.BlockSpec(memory_space=pltpu.VMEM),
            pl.BlockSpec(memory_space=pltpu.VMEM),
        ],
        out_specs=pl.BlockSpec(memory_space=pltpu.VMEM),
        scratch_shapes=[
            pltpu.VMEM((m, d), jnp.bfloat16),
            pltpu.VMEM((m, d), jnp.bfloat16),
            pltpu.SemaphoreType.DMA((C,)),
            pltpu.SemaphoreType.DMA((C,)),
        ],
        compiler_params=pltpu.CompilerParams(collective_id=0),
    )(partial, resid, gamma2)
